# Initial kernel scaffold; baseline (speedup 1.0000x reference)
import jax
import jax.numpy as jnp
from jax.experimental import pallas as pl


def kernel(input_ids, emb_weight):
    raise NotImplementedError("write your pallas kernel here")


# speedup vs baseline: 1.3452x; 1.3452x over previous
import functools

import jax
import jax.numpy as jnp
from jax.experimental import pallas as pl
from jax.experimental.pallas import tpu as pltpu
from jax.experimental.pallas import tpu_sc as plsc

_BIGRAM_VOCAB = 1000000
_MULT = 1009
_DIM = 32
_WINDOW = 128


def _hash_body(ids_ref, out_ref):
    ids = ids_ref[...]
    prev = jnp.concatenate(
        [jnp.zeros((ids.shape[0], 1), jnp.int32), ids[:, :-1]], axis=1
    )
    out_ref[...] = (prev * _MULT + ids) % _BIGRAM_VOCAB


def _bigram_ids(input_ids):
    return pl.pallas_call(
        _hash_body,
        out_shape=jax.ShapeDtypeStruct(input_ids.shape, jnp.int32),
    )(input_ids)


def _sc_gather(table, idx_flat):
    n = idx_flat.shape[0]
    idx2 = idx_flat.reshape(1, n)
    mesh = plsc.VectorSubcoreMesh(core_axis_name="c", subcore_axis_name="s")

    @functools.partial(
        pl.kernel,
        out_type=jax.ShapeDtypeStruct((n, _DIM), jnp.float32),
        mesh=mesh,
        compiler_params=pltpu.CompilerParams(use_tc_tiling_on_sc=False),
    )
    def k(table_hbm, i_hbm, o_hbm):
        def body(i_vmem, o_vmem):
            pltpu.sync_copy(table_hbm.at[i_vmem.at[0]], o_vmem)

        pltpu.emit_pipeline(
            body,
            grid=(n // _WINDOW,),
            in_specs=[pl.BlockSpec((1, _WINDOW), lambda i: (0, i))],
            out_specs=[pl.BlockSpec((_WINDOW, _DIM), lambda i: (i, 0))],
            core_axis_name=("c", "s"),
            dimension_semantics=(pltpu.PARALLEL,),
        )(i_hbm, o_hbm)

    return k(table, idx2)


def kernel(input_ids, emb_weight):
    batch, seq = input_ids.shape
    ids = _bigram_ids(input_ids)
    rows = _sc_gather(emb_weight, ids.reshape(batch * seq))
    return rows.reshape(batch, seq, _DIM)


# speedup vs baseline: 1.4670x; 1.0906x over previous
import functools

import jax
import jax.numpy as jnp
from jax.experimental import pallas as pl
from jax.experimental.pallas import tpu as pltpu
from jax.experimental.pallas import tpu_sc as plsc

_BIGRAM_VOCAB = 1000000
_MULT = 1009
_DIM = 32
_WINDOW = 512


def _hash_body(ids_ref, out_ref):
    ids = ids_ref[...]
    prev = jnp.concatenate(
        [jnp.zeros((ids.shape[0], 1), jnp.int32), ids[:, :-1]], axis=1
    )
    out_ref[...] = (prev * _MULT + ids) % _BIGRAM_VOCAB


def _bigram_ids(input_ids):
    return pl.pallas_call(
        _hash_body,
        out_shape=jax.ShapeDtypeStruct(input_ids.shape, jnp.int32),
    )(input_ids)


def _sc_gather(table, idx_flat):
    n = idx_flat.shape[0]
    idx2 = idx_flat.reshape(1, n)
    mesh = plsc.VectorSubcoreMesh(core_axis_name="c", subcore_axis_name="s")

    @functools.partial(
        pl.kernel,
        out_type=jax.ShapeDtypeStruct((n, _DIM), jnp.float32),
        mesh=mesh,
        compiler_params=pltpu.CompilerParams(use_tc_tiling_on_sc=False),
    )
    def k(table_hbm, i_hbm, o_hbm):
        def body(i_vmem, o_vmem):
            pltpu.sync_copy(table_hbm.at[i_vmem.at[0]], o_vmem)

        pltpu.emit_pipeline(
            body,
            grid=(n // _WINDOW,),
            in_specs=[pl.BlockSpec((1, _WINDOW), lambda i: (0, i))],
            out_specs=[pl.BlockSpec((_WINDOW, _DIM), lambda i: (i, 0))],
            core_axis_name=("c", "s"),
            dimension_semantics=(pltpu.PARALLEL,),
        )(i_hbm, o_hbm)

    return k(table, idx2)


def kernel(input_ids, emb_weight):
    batch, seq = input_ids.shape
    ids = _bigram_ids(input_ids)
    rows = _sc_gather(emb_weight, ids.reshape(batch * seq))
    return rows.reshape(batch, seq, _DIM)


# speedup vs baseline: 1.4909x; 1.0163x over previous
import functools

import jax
import jax.numpy as jnp
from jax.experimental import pallas as pl
from jax.experimental.pallas import tpu as pltpu
from jax.experimental.pallas import tpu_sc as plsc

_BIGRAM_VOCAB = 1000000
_MULT = 1009
_DIM = 32
_WINDOW = 1024


def _hash_body(ids_ref, out_ref):
    ids = ids_ref[...]
    prev = jnp.concatenate(
        [jnp.zeros((ids.shape[0], 1), jnp.int32), ids[:, :-1]], axis=1
    )
    out_ref[...] = (prev * _MULT + ids) % _BIGRAM_VOCAB


def _bigram_ids(input_ids):
    return pl.pallas_call(
        _hash_body,
        out_shape=jax.ShapeDtypeStruct(input_ids.shape, jnp.int32),
    )(input_ids)


def _sc_gather(table, idx_flat):
    n = idx_flat.shape[0]
    idx2 = idx_flat.reshape(1, n)
    mesh = plsc.VectorSubcoreMesh(core_axis_name="c", subcore_axis_name="s")

    @functools.partial(
        pl.kernel,
        out_type=jax.ShapeDtypeStruct((n, _DIM), jnp.float32),
        mesh=mesh,
        compiler_params=pltpu.CompilerParams(use_tc_tiling_on_sc=False),
    )
    def k(table_hbm, i_hbm, o_hbm):
        def body(i_vmem, o_vmem):
            pltpu.sync_copy(table_hbm.at[i_vmem.at[0]], o_vmem)

        pltpu.emit_pipeline(
            body,
            grid=(n // _WINDOW,),
            in_specs=[pl.BlockSpec((1, _WINDOW), lambda i: (0, i))],
            out_specs=[pl.BlockSpec((_WINDOW, _DIM), lambda i: (i, 0))],
            core_axis_name=("c", "s"),
            dimension_semantics=(pltpu.PARALLEL,),
        )(i_hbm, o_hbm)

    return k(table, idx2)


def kernel(input_ids, emb_weight):
    batch, seq = input_ids.shape
    ids = _bigram_ids(input_ids)
    rows = _sc_gather(emb_weight, ids.reshape(batch * seq))
    return rows.reshape(batch, seq, _DIM)
